# baseline (speedup 1.0000x reference)
import functools

import jax
import jax.numpy as jnp
from jax import lax
from jax.experimental import pallas as pl
from jax.experimental.pallas import tpu as pltpu
from jax.experimental.pallas import tpu_sc as plsc

MM = 1000000
DD = 64
BB = 16384
NC, NS, LL = 2, 16, 16
NW = NC * NS
RPW = MM // NW
IPW = BB // (NW * 128)
NCH = 8
CCH = 10
CROWS = RPW // CCH

_mesh = plsc.VectorSubcoreMesh(core_axis_name="c", subcore_axis_name="s")


@functools.partial(
    pl.kernel,
    mesh=_mesh,
    out_type=jax.ShapeDtypeStruct((BB // 128, 128, DD), jnp.float32),
    compiler_params=pltpu.CompilerParams(use_tc_tiling_on_sc=False, needs_layout_passes=False),
    scratch_types=[
        pltpu.VMEM((IPW, 128), jnp.int32),
        pltpu.VMEM((IPW, 128, DD), jnp.float32),
        pltpu.SemaphoreType.DMA,
    ],
)
def _sc_gather(table, ids2d, out, idx_v, rows_v, sem):
    wid = lax.axis_index("s") * NC + lax.axis_index("c")
    base = wid * IPW
    pltpu.sync_copy(ids2d.at[pl.ds(base, IPW)], idx_v)
    cps = [
        pltpu.async_copy(table.at[idx_v.at[j]], rows_v.at[j], sem)
        for j in range(IPW)
    ]
    for c in cps:
        c.wait()
    pltpu.sync_copy(rows_v, out.at[pl.ds(base, IPW)])


def _gru_body(h_ref, m_ref, wit_ref, wht_ref, bi_ref, bh_ref, o_ref):
    h = h_ref[...]
    gi = jnp.dot(m_ref[...], wit_ref[...], preferred_element_type=jnp.float32)
    gi = gi + bi_ref[...]
    gh = jnp.dot(h, wht_ref[...], preferred_element_type=jnp.float32)
    gh = gh + bh_ref[...]
    r = jax.nn.sigmoid(gi[:, 0:DD] + gh[:, 0:DD])
    z = jax.nn.sigmoid(gi[:, DD:2 * DD] + gh[:, DD:2 * DD])
    n = jnp.tanh(gi[:, 2 * DD:3 * DD] + r * gh[:, 2 * DD:3 * DD])
    o_ref[...] = (1.0 - z) * n + z * h


_GRU_BLK = 2048
_gru_call = pl.pallas_call(
    _gru_body,
    grid=(BB // _GRU_BLK,),
    in_specs=[
        pl.BlockSpec((_GRU_BLK, DD), lambda i: (i, 0)),
        pl.BlockSpec((_GRU_BLK, DD), lambda i: (i, 0)),
        pl.BlockSpec((DD, 3 * DD), lambda i: (0, 0)),
        pl.BlockSpec((DD, 3 * DD), lambda i: (0, 0)),
        pl.BlockSpec((1, 3 * DD), lambda i: (0, 0)),
        pl.BlockSpec((1, 3 * DD), lambda i: (0, 0)),
    ],
    out_specs=pl.BlockSpec((_GRU_BLK, DD), lambda i: (i, 0)),
    out_shape=jax.ShapeDtypeStruct((BB, DD), jnp.float32),
)


@functools.partial(
    pl.kernel,
    mesh=_mesh,
    out_type=jax.ShapeDtypeStruct((MM, DD), jnp.float32),
    compiler_params=pltpu.CompilerParams(use_tc_tiling_on_sc=False, needs_layout_passes=False),
    scratch_types=[
        pltpu.VMEM((BB,), jnp.int32),
        pltpu.VMEM((NCH, 128), jnp.int32),
        pltpu.VMEM((NCH, 128), jnp.int32),
        pltpu.VMEM((128, DD), jnp.float32),
        pltpu.SemaphoreType.DMA,
        pltpu.SemaphoreType.DMA,
        pltpu.SemaphoreType.DMA,
    ],
)
def _sc_copy_scatter(table, ids, upd, out, ids_v, gidx, sidx, rows_v,
                     csem, gsem, ssem):
    wid = lax.axis_index("s") * NC + lax.axis_index("c")
    base = wid * RPW

    cps = [
        pltpu.async_copy(
            table.at[pl.ds(base + k * CROWS, CROWS)],
            out.at[pl.ds(base + k * CROWS, CROWS)],
            csem,
        )
        for k in range(CCH)
    ]

    pltpu.sync_copy(ids, ids_v)

    ids0 = plsc.load_gather(ids_v, [jnp.zeros((LL,), jnp.int32)])
    zeros = jnp.zeros((LL,), jnp.int32)
    for j in range(NCH):
        for kk in range(128 // LL):
            gidx[j, pl.ds(kk * LL, LL)] = zeros
            sidx[j, pl.ds(kk * LL, LL)] = ids0

    def scan_body(i, cnt):
        idv = ids_v[pl.ds(i * LL, LL)]
        m = (idv >= base) & (idv < base + RPW)
        mi = m.astype(jnp.int32)
        pos = cnt + plsc.cumsum(mi) - 1
        pos = jnp.where(m, pos, 0)
        row = pos // 128
        col = pos - row * 128
        bpos = lax.iota(jnp.int32, LL) + i * LL
        plsc.store_scatter(gidx, [row, col], bpos, mask=m)
        plsc.store_scatter(sidx, [row, col], idv, mask=m)
        return cnt + jnp.sum(mi)

    cnt = lax.fori_loop(0, BB // LL, scan_body, jnp.int32(0))

    for c in cps:
        c.wait()

    for j in range(NCH):
        @pl.when(j * 128 < cnt)
        def _():
            pltpu.async_copy(upd.at[gidx.at[j]], rows_v, gsem).wait()
            pltpu.async_copy(rows_v, out.at[sidx.at[j]], ssem).wait()


def kernel(node_memories, node_ids, messages, W_ih, W_hh, b_ih, b_hh):
    ids2d = node_ids.reshape(BB // 128, 128)
    h = _sc_gather(node_memories, ids2d).reshape(BB, DD)
    upd = _gru_call(
        h,
        messages,
        W_ih.T,
        W_hh.T,
        b_ih.reshape(1, 3 * DD),
        b_hh.reshape(1, 3 * DD),
    )
    return _sc_copy_scatter(node_memories, node_ids, upd)


# speedup vs baseline: 4.1854x; 4.1854x over previous
import functools

import jax
import jax.numpy as jnp
from jax import lax
from jax.experimental import pallas as pl
from jax.experimental.pallas import tpu as pltpu
from jax.experimental.pallas import tpu_sc as plsc

MM = 1000000
DD = 64
BB = 16384
NC, NS, LL = 2, 16, 16
NW = NC * NS
RPW = MM // NW
IPW = BB // (NW * 128)
NCH = 8
CCH = 10
CROWS = RPW // CCH

_mesh = plsc.VectorSubcoreMesh(core_axis_name="c", subcore_axis_name="s")


@functools.partial(
    pl.kernel,
    mesh=_mesh,
    out_type=jax.ShapeDtypeStruct((BB // 128, 128, DD), jnp.float32),
    compiler_params=pltpu.CompilerParams(use_tc_tiling_on_sc=False, needs_layout_passes=False),
    scratch_types=[
        pltpu.VMEM((IPW, 128), jnp.int32),
        pltpu.VMEM((IPW, 128, DD), jnp.float32),
        pltpu.SemaphoreType.DMA,
    ],
)
def _sc_gather(table, ids2d, out, idx_v, rows_v, sem):
    wid = lax.axis_index("s") * NC + lax.axis_index("c")
    base = wid * IPW
    pltpu.sync_copy(ids2d.at[pl.ds(base, IPW)], idx_v)
    cps = [
        pltpu.async_copy(table.at[idx_v.at[j]], rows_v.at[j], sem)
        for j in range(IPW)
    ]
    for c in cps:
        c.wait()
    pltpu.sync_copy(rows_v, out.at[pl.ds(base, IPW)])


def _gru_body(h_ref, m_ref, wit_ref, wht_ref, bi_ref, bh_ref, o_ref):
    h = h_ref[...]
    gi = jnp.dot(m_ref[...], wit_ref[...], preferred_element_type=jnp.float32)
    gi = gi + bi_ref[...]
    gh = jnp.dot(h, wht_ref[...], preferred_element_type=jnp.float32)
    gh = gh + bh_ref[...]
    r = jax.nn.sigmoid(gi[:, 0:DD] + gh[:, 0:DD])
    z = jax.nn.sigmoid(gi[:, DD:2 * DD] + gh[:, DD:2 * DD])
    n = jnp.tanh(gi[:, 2 * DD:3 * DD] + r * gh[:, 2 * DD:3 * DD])
    o_ref[...] = (1.0 - z) * n + z * h


_GRU_BLK = 2048
_gru_call = pl.pallas_call(
    _gru_body,
    grid=(BB // _GRU_BLK,),
    in_specs=[
        pl.BlockSpec((_GRU_BLK, DD), lambda i: (i, 0)),
        pl.BlockSpec((_GRU_BLK, DD), lambda i: (i, 0)),
        pl.BlockSpec((DD, 3 * DD), lambda i: (0, 0)),
        pl.BlockSpec((DD, 3 * DD), lambda i: (0, 0)),
        pl.BlockSpec((1, 3 * DD), lambda i: (0, 0)),
        pl.BlockSpec((1, 3 * DD), lambda i: (0, 0)),
    ],
    out_specs=pl.BlockSpec((_GRU_BLK, DD), lambda i: (i, 0)),
    out_shape=jax.ShapeDtypeStruct((BB, DD), jnp.float32),
)


_COPY_BLK = 8000
_copy_body = lambda i_ref, o_ref: o_ref.__setitem__((...,), i_ref[...])
_copy_call = pl.pallas_call(
    _copy_body,
    grid=(MM // _COPY_BLK,),
    in_specs=[pl.BlockSpec((_COPY_BLK, DD), lambda i: (i, 0))],
    out_specs=pl.BlockSpec((_COPY_BLK, DD), lambda i: (i, 0)),
    out_shape=jax.ShapeDtypeStruct((MM, DD), jnp.float32),
)


@functools.partial(
    pl.kernel,
    mesh=_mesh,
    out_type=(),
    compiler_params=pltpu.CompilerParams(use_tc_tiling_on_sc=False, needs_layout_passes=False),
    scratch_types=[
        pltpu.VMEM((BB,), jnp.int32),
        pltpu.VMEM((NCH, 128), jnp.int32),
        pltpu.VMEM((NCH, 128), jnp.int32),
        pltpu.VMEM((128, DD), jnp.float32),
        pltpu.SemaphoreType.DMA,
        pltpu.SemaphoreType.DMA,
    ],
)
def _sc_scatter(out, ids, upd, ids_v, gidx, sidx, rows_v, gsem, ssem):
    wid = lax.axis_index("s") * NC + lax.axis_index("c")
    base = wid * RPW

    pltpu.sync_copy(ids, ids_v)

    ids0 = plsc.load_gather(ids_v, [jnp.zeros((LL,), jnp.int32)])
    zeros = jnp.zeros((LL,), jnp.int32)
    for j in range(NCH):
        for kk in range(128 // LL):
            gidx[j, pl.ds(kk * LL, LL)] = zeros
            sidx[j, pl.ds(kk * LL, LL)] = ids0

    def scan_body(i, cnt):
        idv = ids_v[pl.ds(i * LL, LL)]
        m = (idv >= base) & (idv < base + RPW)
        mi = m.astype(jnp.int32)
        pos = cnt + plsc.cumsum(mi) - 1
        pos = jnp.where(m, pos, 0)
        row = pos // 128
        col = pos - row * 128
        bpos = lax.iota(jnp.int32, LL) + i * LL
        plsc.store_scatter(gidx, [row, col], bpos, mask=m)
        plsc.store_scatter(sidx, [row, col], idv, mask=m)
        return cnt + jnp.sum(mi)

    cnt = lax.fori_loop(0, BB // LL, scan_body, jnp.int32(0), unroll=8)

    for j in range(NCH):
        @pl.when(j * 128 < cnt)
        def _():
            pltpu.async_copy(upd.at[gidx.at[j]], rows_v, gsem).wait()
            pltpu.async_copy(rows_v, out.at[sidx.at[j]], ssem).wait()


def kernel(node_memories, node_ids, messages, W_ih, W_hh, b_ih, b_hh):
    ids2d = node_ids.reshape(BB // 128, 128)
    h = _sc_gather(node_memories, ids2d).reshape(BB, DD)
    upd = _gru_call(
        h,
        messages,
        W_ih.T,
        W_hh.T,
        b_ih.reshape(1, 3 * DD),
        b_hh.reshape(1, 3 * DD),
    )
    table_ref = jax.new_ref(_copy_call(node_memories))
    _sc_scatter(table_ref, node_ids, upd)
    return jax.freeze(table_ref)
